# baseline (speedup 1.0000x reference)
import functools

import jax
import jax.numpy as jnp
from jax import lax
from jax.experimental import pallas as pl
from jax.experimental.pallas import tpu as pltpu
from jax.experimental.pallas import tpu_sc as plsc

N0, N1, N2 = 12288, 3072, 768
KNN = 10
B = 4
RB = 512
NW = 32


_LAP_CACHE = {}


def _make_lap(N, D, sub):
    key = (N, D, sub)
    if key in _LAP_CACHE:
        return _LAP_CACHE[key]
    nt = N // NW
    EA = nt * KNN
    EC = 80 if (D <= 512 and (EA // 80) % 2 == 0) else 40
    GN = EC // KNN
    ngc = EA // EC
    mesh = plsc.VectorSubcoreMesh(core_axis_name="c", subcore_axis_name="s")

    scratch = [
        pltpu.VMEM((EA + 2 * EC,), jnp.int32),
        pltpu.VMEM((EA,), jnp.float32),
        pltpu.VMEM((EC, D), jnp.float32),
        pltpu.VMEM((EC, D), jnp.float32),
        pltpu.VMEM((8, D), jnp.float32),
        pltpu.SemaphoreType.DMA,
        pltpu.SemaphoreType.DMA,
    ]
    if sub:
        scratch.append(pltpu.VMEM((8, D), jnp.float32))

    def body(x_hbm, src_hbm, w_hbm, *rest):
        if sub:
            (x0_hbm, out_hbm, idx_all, w_all, ga, gb,
             o_v, sem_a, sem_b, x0_v) = rest
        else:
            (out_hbm, idx_all, w_all, ga, gb,
             o_v, sem_a, sem_b) = rest
        bufs = (ga, gb)
        sems = (sem_a, sem_b)
        wid = lax.axis_index("s") * 2 + lax.axis_index("c")
        base_n = wid * nt
        base_e = wid * EA

        pltpu.sync_copy(src_hbm.at[pl.ds(base_e, EA)],
                        idx_all.at[pl.ds(0, EA)])
        pltpu.sync_copy(w_hbm.at[pl.ds(base_e, EA)], w_all)
        z16 = jnp.zeros((16,), jnp.int32)
        for j in range(2 * EC // 16):
            idx_all[pl.ds(EA + 16 * j, 16)] = z16

        def gstart(gi, b):
            pltpu.make_async_copy(
                x_hbm.at[idx_all.at[pl.ds(gi * EC, EC)]],
                bufs[b], sems[b]).start()

        def gwait(b):
            pltpu.make_async_copy(
                x_hbm.at[idx_all.at[pl.ds(0, EC)]],
                bufs[b], sems[b]).wait()

        def compute8(oc0, buf_of_n, row_of_n):
            wbase = 80 * oc0
            wvecs = [w_all[pl.ds(wbase + 16 * j, 16)] for j in range(5)]
            for n in range(8):
                buf = buf_of_n(n)
                r0 = row_of_n(n)
                ws = [wvecs[(n * KNN + k) // 16][(n * KNN + k) % 16]
                      for k in range(KNN)]

                def dbody(di, c2):
                    col = di * 16
                    acc = ws[0] * buf[r0, pl.ds(col, 16)]
                    for k in range(1, KNN):
                        acc = acc + ws[k] * buf[r0 + k, pl.ds(col, 16)]
                    if sub:
                        o_v[n, pl.ds(col, 16)] = (
                            2.0 * acc - x0_v[n, pl.ds(col, 16)])
                    else:
                        o_v[n, pl.ds(col, 16)] = acc
                    return c2

                lax.fori_loop(0, D // 16, dbody, 0)

        gstart(0, 0)
        gstart(1, 1)

        if GN == 8:
            def pair8(pi, carry):
                for b in range(2):
                    oc = 2 * pi + b
                    node0 = base_n + oc * 8
                    if sub:
                        pltpu.sync_copy(x0_hbm.at[pl.ds(node0, 8)], x0_v)
                    gwait(b)
                    compute8(oc, lambda n: bufs[b], lambda n: KNN * n)
                    gstart(oc + 2, b)
                    pltpu.sync_copy(o_v, out_hbm.at[pl.ds(node0, 8)])
                return carry

            lax.fori_loop(0, ngc // 2, pair8, 0)
        else:
            def pair4(pi, carry):
                node0 = base_n + pi * 8
                wbase = 80 * pi
                wvecs = [w_all[pl.ds(wbase + 16 * j, 16)]
                         for j in range(5)]
                if sub:
                    pltpu.sync_copy(x0_hbm.at[pl.ds(node0, 8)], x0_v)

                def cnodes(nlo, buf):
                    for n in range(nlo, nlo + 4):
                        r0 = KNN * (n - nlo)
                        ws = [wvecs[(n * KNN + k) // 16][(n * KNN + k) % 16]
                              for k in range(KNN)]

                        def dbody(di, c2):
                            col = di * 16
                            acc = ws[0] * buf[r0, pl.ds(col, 16)]
                            for k in range(1, KNN):
                                acc = acc + ws[k] * buf[r0 + k,
                                                        pl.ds(col, 16)]
                            if sub:
                                o_v[n, pl.ds(col, 16)] = (
                                    2.0 * acc - x0_v[n, pl.ds(col, 16)])
                            else:
                                o_v[n, pl.ds(col, 16)] = acc
                            return c2

                        lax.fori_loop(0, D // 16, dbody, 0)

                gwait(0)
                cnodes(0, bufs[0])
                gstart(2 * pi + 2, 0)
                gwait(1)
                cnodes(4, bufs[1])
                gstart(2 * pi + 3, 1)
                pltpu.sync_copy(o_v, out_hbm.at[pl.ds(node0, 8)])
                return carry

            lax.fori_loop(0, ngc // 2, pair4, 0)

        gwait(0)
        gwait(1)

    lap_k = functools.partial(
        pl.kernel, mesh=mesh,
        out_type=jax.ShapeDtypeStruct((N, D), jnp.float32),
        scratch_types=scratch,
    )(body)
    _LAP_CACHE[key] = lap_k
    return lap_k


def _lap(x_nd, src, w):
    N, D = x_nd.shape
    return _make_lap(N, D, False)(x_nd, src, w)


def _lap_sub(x_nd, src, w, x0_nd):
    N, D = x_nd.shape
    return _make_lap(N, D, True)(x_nd, src, w, x0_nd)


def _mm(terms, addends=(), bias8=None, stats=False):
    if terms:
        R = terms[0][0].shape[0]
        cout = terms[0][1].shape[1]
    else:
        R = addends[0][1].shape[0]
        cout = addends[0][1].shape[1]
    nt_ = len(terms)

    args = []
    in_specs = []
    for x, W in terms:
        args.append(x)
        in_specs.append(pl.BlockSpec((RB, x.shape[1]), lambda i: (i, 0)))
        args.append(W)
        in_specs.append(
            pl.BlockSpec((W.shape[0], W.shape[1]), lambda i: (0, 0)))
    for a in addends:
        args.append(a[1])
        in_specs.append(pl.BlockSpec((RB, cout), lambda i: (i, 0)))
        if a[0] == "bnrelu":
            args.append(a[2])
            in_specs.append(pl.BlockSpec((8, cout), lambda i: (0, 0)))
    if bias8 is not None:
        args.append(bias8)
        in_specs.append(pl.BlockSpec((8, cout), lambda i: (0, 0)))

    def body(*refs):
        it = iter(refs)
        acc = None
        for _ in range(nt_):
            xr = next(it)
            wr = next(it)
            d = lax.dot_general(xr[...], wr[...], (((1,), (0,)), ((), ())),
                                preferred_element_type=jnp.float32)
            acc = d if acc is None else acc + d
        for a in addends:
            ar = next(it)
            if a[0] == "plain":
                term = ar[...] if a[2] == 1.0 else ar[...] * a[2]
            else:
                aff = next(it)
                term = jnp.maximum(
                    ar[...] * aff[0:1, :] + aff[1:2, :], 0.0)
            acc = term if acc is None else acc + term
        if bias8 is not None:
            br = next(it)
            acc = acc + br[0:1, :]
        y_ref = next(it)
        y_ref[...] = acc
        if stats:
            st_ref = next(it)
            s = jnp.sum(acc, axis=0, keepdims=True)
            ss = jnp.sum(acc * acc, axis=0, keepdims=True)
            blk = jnp.concatenate(
                [s, ss, jnp.zeros((6, acc.shape[1]), jnp.float32)], axis=0)

            @pl.when(pl.program_id(0) == 0)
            def _():
                st_ref[...] = blk

            @pl.when(pl.program_id(0) > 0)
            def _():
                st_ref[...] += blk

    out_shape = [jax.ShapeDtypeStruct((R, cout), jnp.float32)]
    out_specs = [pl.BlockSpec((RB, cout), lambda i: (i, 0))]
    if stats:
        out_shape.append(jax.ShapeDtypeStruct((8, cout), jnp.float32))
        out_specs.append(pl.BlockSpec((8, cout), lambda i: (0, 0)))

    res = pl.pallas_call(
        body,
        grid=(R // RB,),
        in_specs=in_specs,
        out_specs=out_specs,
        out_shape=out_shape,
    )(*args)
    return res if stats else res[0]


def _affine8(st, g, be, R):
    m = st[0] / R
    v = st[1] / R - m * m
    scale = g * lax.rsqrt(v + 1e-5)
    shift = be - m * scale
    return jnp.concatenate(
        [scale[None], shift[None],
         jnp.zeros((6, scale.shape[0]), jnp.float32)], axis=0)


def _bias8(b):
    return jnp.concatenate(
        [b[None], jnp.zeros((7, b.shape[0]), jnp.float32)], axis=0)


def _pool4(x, N, C):
    M = N // 4
    NB = 128
    while M % NB:
        NB //= 2

    def body(x_ref, mx_ref, idx_ref):
        xb = x_ref[...].reshape(NB, 4, B, C)
        best = xb[:, 0]
        idx = jnp.zeros((NB, B, C), jnp.int32)
        for j in range(1, 4):
            xj = xb[:, j]
            better = xj > best
            idx = jnp.where(better, j, idx)
            best = jnp.where(better, xj, best)
        mx_ref[...] = best.reshape(NB * B, C)
        idx_ref[...] = idx.reshape(NB * B, C)

    return pl.pallas_call(
        body,
        grid=(M // NB,),
        in_specs=[pl.BlockSpec((NB * 4 * B, C), lambda i: (i, 0))],
        out_specs=[pl.BlockSpec((NB * B, C), lambda i: (i, 0)),
                   pl.BlockSpec((NB * B, C), lambda i: (i, 0))],
        out_shape=[jax.ShapeDtypeStruct((M * B, C), jnp.float32),
                   jax.ShapeDtypeStruct((M * B, C), jnp.int32)],
    )(x)


def _unpool4(y, idx, M, C):
    NB = 128
    while M % NB:
        NB //= 2

    def body(y_ref, i_ref, o_ref):
        yr = y_ref[...].reshape(NB, 1, B, C)
        ir = i_ref[...].reshape(NB, 1, B, C)
        j = lax.broadcasted_iota(jnp.int32, (NB, 4, B, C), 1)
        out = jnp.where(ir == j, yr, 0.0)
        o_ref[...] = out.reshape(NB * 4 * B, C)

    return pl.pallas_call(
        body,
        grid=(M // NB,),
        in_specs=[pl.BlockSpec((NB * B, C), lambda i: (i, 0)),
                  pl.BlockSpec((NB * B, C), lambda i: (i, 0))],
        out_specs=pl.BlockSpec((NB * 4 * B, C), lambda i: (i, 0)),
        out_shape=jax.ShapeDtypeStruct((M * 4 * B, C), jnp.float32),
    )(y, idx)


def _split3(W):
    return jnp.split(W, 3, axis=0)


def _cheb(xs, src, w, Ws, N, stats=True, bias8=None, extra=()):
    terms = []
    for x, Wi in zip(xs, Ws):
        W0, W1, W2 = _split3(Wi)
        cin = x.shape[1]
        xv = x.reshape(N, B * cin)
        x1 = _lap(xv, src, w)
        x2 = _lap_sub(x1, src, w, xv).reshape(N * B, cin)
        x1 = x1.reshape(N * B, cin)
        terms += [(x, W0), (x1, W1), (x2, W2)]
    return _mm(terms, list(extra), bias8=bias8, stats=stats)


def kernel(x, params, lap_w, edges):
    p = params
    s0, w0 = edges["src0"], lap_w["w0"]
    s1, w1 = edges["src1"], lap_w["w1"]
    s2, w2 = edges["src2"], lap_w["w2"]

    xa = jnp.transpose(x, (2, 0, 1, 3)).reshape(N0, B, 14)
    xa = jnp.pad(xa, ((0, 0), (0, 0), (0, 18))).reshape(N0 * B, 32)

    padW = lambda W: jnp.pad(W, ((0, 18), (0, 0)))

    def splitW(W, cs):
        W0, W1, W2 = _split3(W)
        out, o = [], 0
        for c in cs:
            out.append(jnp.concatenate(
                [W0[o:o + c], W1[o:o + c], W2[o:o + c]], axis=0))
            o += c
        return out

    W0, W1, W2 = _split3(p["c11"]["W"])
    Wc11 = jnp.concatenate([padW(W0), padW(W1), padW(W2)], axis=0)
    y, st = _cheb([xa], s0, w0, [Wc11], N0)
    e11 = _mm([], [("bnrelu", y,
                    _affine8(st, p["c11"]["g"], p["c11"]["be"], N0 * B))])

    y, st = _cheb([e11], s0, w0, [p["c13"]["W"]], N0)
    aff = _affine8(st, p["c13"]["g"], p["c13"]["be"], N0 * B)
    e1 = _mm([(xa, padW(p["r1"]["W"]))], [("bnrelu", y, aff)],
             bias8=_bias8(p["r1"]["b"]))

    e2i, idx1 = _pool4(e1, N0, 128)

    y, st = _cheb([e2i], s1, w1, [p["c21"]["W"]], N1)
    e2a = _mm([], [("bnrelu", y,
                    _affine8(st, p["c21"]["g"], p["c21"]["be"], N1 * B))])
    y, st = _cheb([e2a], s1, w1, [p["c23"]["W"]], N1)
    aff = _affine8(st, p["c23"]["g"], p["c23"]["be"], N1 * B)
    e2 = _mm([(e2i, p["r2"]["W"])], [("bnrelu", y, aff)],
             bias8=_bias8(p["r2"]["b"]))

    e3i, idx2 = _pool4(e2, N1, 256)

    y, st = _cheb([e3i], s2, w2, [p["c31"]["W"]], N2)
    e3a = _mm([], [("bnrelu", y,
                    _affine8(st, p["c31"]["g"], p["c31"]["be"], N2 * B))])
    y, st = _cheb([e3a[:, :256], e3a[:, 256:]], s2, w2,
                  splitW(p["c33"]["W"], [256, 256]), N2)
    aff = _affine8(st, p["c33"]["g"], p["c33"]["be"], N2 * B)
    e3 = _mm([(e3i, p["r3"]["W"])], [("bnrelu", y, aff)],
             bias8=_bias8(p["r3"]["b"]))

    u2 = _unpool4(e3, idx2, N2, 256)
    y, st = _cheb([u2, e2], s1, w1, splitW(p["u21"]["W"], [256, 256]), N1)
    y21 = _mm([], [("bnrelu", y,
                    _affine8(st, p["u21"]["g"], p["u21"]["be"], N1 * B))])
    y, st = _cheb([y21], s1, w1, [p["u22"]["W"]], N1)
    y22 = _mm([], [("bnrelu", y,
                    _affine8(st, p["u22"]["g"], p["u22"]["be"], N1 * B))])

    u1 = _unpool4(y22, idx1, N1, 128)
    y, st = _cheb([u1, e1], s0, w0, splitW(p["u11"]["W"], [128, 128]), N0)
    y11 = _mm([], [("bnrelu", y,
                    _affine8(st, p["u11"]["g"], p["u11"]["be"], N0 * B))])
    y, st = _cheb([y11], s0, w0, [p["u12"]["W"]], N0)
    y12 = _mm([], [("bnrelu", y,
                    _affine8(st, p["u12"]["g"], p["u12"]["be"], N0 * B))])

    xl = jnp.transpose(x[:, 1, :, 5:7], (1, 0, 2))
    xl4 = jnp.concatenate([xl, xl], axis=-1).reshape(N0 * B, 4)
    yf = _cheb([y12, e11], s0, w0, splitW(p["u13"]["W"], [64, 64]), N0,
               stats=False, bias8=_bias8(p["u13"]["b"]),
               extra=[("plain", xl4, 1.0)])

    out = yf.reshape(N0, B, 2, 2)
    return jnp.transpose(out, (1, 2, 0, 3))


# speedup vs baseline: 1.9628x; 1.9628x over previous
import functools

import jax
import jax.numpy as jnp
from jax import lax
from jax.experimental import pallas as pl
from jax.experimental.pallas import tpu as pltpu
from jax.experimental.pallas import tpu_sc as plsc

N0, N1, N2 = 12288, 3072, 768
KNN = 10
B = 4
RB = 512
NW = 32


_LAP_CACHE = {}


def _lap_cfg(N, D):
    nt = N // NW
    EA = nt * KNN
    if D <= 512:
        ec, nbuf = 80, 2
    else:
        ec, nbuf = 40, 2
    while True:
        gn = ec // KNN
        sup = max(nbuf, 8 // gn)
        if (EA // ec) % sup == 0:
            break
        if ec == 80:
            ec = 40
        else:
            nbuf = 1
    return ec, nbuf, False


def _make_lap(N, D, sub, cfg=None):
    cfg = cfg or _lap_cfg(N, D)
    key = (N, D, sub, cfg)
    if key in _LAP_CACHE:
        return _LAP_CACHE[key]
    ec, nbuf, whole_idx = cfg
    nt = N // NW
    EA = nt * KNN
    GN = ec // KNN
    ngc = EA // ec
    CPB = 8 // GN
    SUP = max(nbuf, CPB)
    assert ngc % SUP == 0 and SUP % CPB == 0
    mesh = plsc.VectorSubcoreMesh(core_axis_name="c", subcore_axis_name="s")

    scratch = [pltpu.VMEM((EA,), jnp.float32)]
    if whole_idx:
        scratch.append(pltpu.VMEM((EA + nbuf * ec,), jnp.int32))
    else:
        scratch += [pltpu.VMEM((ec,), jnp.int32) for _ in range(nbuf)]
    scratch += [pltpu.VMEM((ec, D), jnp.float32) for _ in range(nbuf)]
    scratch += [pltpu.SemaphoreType.DMA for _ in range(nbuf)]
    scratch.append(pltpu.VMEM((8, D), jnp.float32))
    if sub:
        scratch.append(pltpu.VMEM((8, D), jnp.float32))

    def body(x_hbm, src_hbm, w_hbm, *rest):
        rest = list(rest)
        if sub:
            x0_hbm = rest.pop(0)
        out_hbm = rest.pop(0)
        w_all = rest.pop(0)
        if whole_idx:
            idx_all = rest.pop(0)
            idxb = None
        else:
            idxb = [rest.pop(0) for _ in range(nbuf)]
        bufs = [rest.pop(0) for _ in range(nbuf)]
        sems = [rest.pop(0) for _ in range(nbuf)]
        o_v = rest.pop(0)
        x0_v = rest.pop(0) if sub else None
        wid = lax.axis_index("s") * 2 + lax.axis_index("c")
        base_n = wid * nt
        base_e = wid * EA

        pltpu.sync_copy(w_hbm.at[pl.ds(base_e, EA)], w_all)
        if whole_idx:
            pltpu.sync_copy(src_hbm.at[pl.ds(base_e, EA)],
                            idx_all.at[pl.ds(0, EA)])
            z16 = jnp.zeros((16,), jnp.int32)
            for j in range(nbuf * ec // 16):
                idx_all[pl.ds(EA + 16 * j, 16)] = z16

        def gissue(gi, b):
            if whole_idx:
                pltpu.make_async_copy(
                    x_hbm.at[idx_all.at[pl.ds(gi * ec, ec)]],
                    bufs[b], sems[b]).start()
            else:
                @pl.when(gi < ngc)
                def _():
                    pltpu.sync_copy(src_hbm.at[pl.ds(base_e + gi * ec, ec)],
                                    idxb[b])
                    pltpu.make_async_copy(
                        x_hbm.at[idxb[b]], bufs[b], sems[b]).start()

        def gwait(b):
            src = (x_hbm.at[idx_all.at[pl.ds(0, ec)]] if whole_idx
                   else x_hbm.at[idxb[b]])
            pltpu.make_async_copy(src, bufs[b], sems[b]).wait()

        def compute_chunk(oc, cb, buf):
            wbase = 80 * oc
            wvecs = [w_all[pl.ds(wbase + 16 * j, 16)] for j in range(5)]
            for nn in range(GN):
                n = cb * GN + nn
                r0 = KNN * nn
                e = n * KNN
                ws = [wvecs[(e + k) // 16][(e + k) % 16]
                      for k in range(KNN)]

                def dbody(di, c2):
                    col = di * 16
                    p = [ws[k] * buf[r0 + k, pl.ds(col, 16)]
                         for k in range(KNN)]
                    while len(p) > 1:
                        p = [p[i] + p[i + 1] for i in range(0, len(p) - 1, 2)] \
                            + ([p[-1]] if len(p) % 2 else [])
                    if sub:
                        o_v[n, pl.ds(col, 16)] = (
                            2.0 * p[0] - x0_v[n, pl.ds(col, 16)])
                    else:
                        o_v[n, pl.ds(col, 16)] = p[0]
                    return c2

                lax.fori_loop(0, D // 16, dbody, 0)

        for b in range(nbuf):
            gissue(b, b)

        def super_iter(si, carry):
            for j in range(SUP):
                gi = si * SUP + j
                cb = j % CPB
                oc = si * (SUP // CPB) + j // CPB
                if sub and cb == 0:
                    pltpu.sync_copy(
                        x0_hbm.at[pl.ds(base_n + oc * 8, 8)], x0_v)
                b = j % nbuf
                gwait(b)
                compute_chunk(oc, cb, bufs[b])
                gissue(gi + nbuf, b)
                if cb == CPB - 1:
                    pltpu.sync_copy(o_v, out_hbm.at[pl.ds(base_n + oc * 8, 8)])
            return carry

        lax.fori_loop(0, ngc // SUP, super_iter, 0)

        if whole_idx:
            for b in range(nbuf):
                gwait(b)

    lap_k = functools.partial(
        pl.kernel, mesh=mesh,
        out_type=jax.ShapeDtypeStruct((N, D), jnp.float32),
        scratch_types=scratch,
    )(body)
    _LAP_CACHE[key] = lap_k
    return lap_k


def _lap(x_nd, src, w):
    N, D = x_nd.shape
    return _make_lap(N, D, False)(x_nd, src, w)


def _lap_sub(x_nd, src, w, x0_nd):
    N, D = x_nd.shape
    return _make_lap(N, D, True)(x_nd, src, w, x0_nd)


def _mm(terms, addends=(), bias8=None, stats=False):
    if terms:
        R = terms[0][0].shape[0]
        cout = terms[0][1].shape[1]
    else:
        R = addends[0][1].shape[0]
        cout = addends[0][1].shape[1]
    nt_ = len(terms)

    args = []
    in_specs = []
    for x, W in terms:
        args.append(x)
        in_specs.append(pl.BlockSpec((RB, x.shape[1]), lambda i: (i, 0)))
        args.append(W)
        in_specs.append(
            pl.BlockSpec((W.shape[0], W.shape[1]), lambda i: (0, 0)))
    for a in addends:
        args.append(a[1])
        in_specs.append(pl.BlockSpec((RB, cout), lambda i: (i, 0)))
        if a[0] == "bnrelu":
            args.append(a[2])
            in_specs.append(pl.BlockSpec((8, cout), lambda i: (0, 0)))
    if bias8 is not None:
        args.append(bias8)
        in_specs.append(pl.BlockSpec((8, cout), lambda i: (0, 0)))

    def body(*refs):
        it = iter(refs)
        acc = None
        for _ in range(nt_):
            xr = next(it)
            wr = next(it)
            d = lax.dot_general(xr[...], wr[...], (((1,), (0,)), ((), ())),
                                preferred_element_type=jnp.float32)
            acc = d if acc is None else acc + d
        for a in addends:
            ar = next(it)
            if a[0] == "plain":
                term = ar[...] if a[2] == 1.0 else ar[...] * a[2]
            else:
                aff = next(it)
                term = jnp.maximum(
                    ar[...] * aff[0:1, :] + aff[1:2, :], 0.0)
            acc = term if acc is None else acc + term
        if bias8 is not None:
            br = next(it)
            acc = acc + br[0:1, :]
        y_ref = next(it)
        y_ref[...] = acc
        if stats:
            st_ref = next(it)
            s = jnp.sum(acc, axis=0, keepdims=True)
            ss = jnp.sum(acc * acc, axis=0, keepdims=True)
            blk = jnp.concatenate(
                [s, ss, jnp.zeros((6, acc.shape[1]), jnp.float32)], axis=0)

            @pl.when(pl.program_id(0) == 0)
            def _():
                st_ref[...] = blk

            @pl.when(pl.program_id(0) > 0)
            def _():
                st_ref[...] += blk

    out_shape = [jax.ShapeDtypeStruct((R, cout), jnp.float32)]
    out_specs = [pl.BlockSpec((RB, cout), lambda i: (i, 0))]
    if stats:
        out_shape.append(jax.ShapeDtypeStruct((8, cout), jnp.float32))
        out_specs.append(pl.BlockSpec((8, cout), lambda i: (0, 0)))

    res = pl.pallas_call(
        body,
        grid=(R // RB,),
        in_specs=in_specs,
        out_specs=out_specs,
        out_shape=out_shape,
    )(*args)
    return res if stats else res[0]


def _affine8(st, g, be, R):
    m = st[0] / R
    v = st[1] / R - m * m
    scale = g * lax.rsqrt(v + 1e-5)
    shift = be - m * scale
    return jnp.concatenate(
        [scale[None], shift[None],
         jnp.zeros((6, scale.shape[0]), jnp.float32)], axis=0)


def _bias8(b):
    return jnp.concatenate(
        [b[None], jnp.zeros((7, b.shape[0]), jnp.float32)], axis=0)


def _pool4(x, N, C):
    M = N // 4
    NB = 128
    while M % NB:
        NB //= 2

    def body(x_ref, mx_ref, idx_ref):
        xb = x_ref[...].reshape(NB, 4, B, C)
        best = xb[:, 0]
        idx = jnp.zeros((NB, B, C), jnp.int32)
        for j in range(1, 4):
            xj = xb[:, j]
            better = xj > best
            idx = jnp.where(better, j, idx)
            best = jnp.where(better, xj, best)
        mx_ref[...] = best.reshape(NB * B, C)
        idx_ref[...] = idx.reshape(NB * B, C)

    return pl.pallas_call(
        body,
        grid=(M // NB,),
        in_specs=[pl.BlockSpec((NB * 4 * B, C), lambda i: (i, 0))],
        out_specs=[pl.BlockSpec((NB * B, C), lambda i: (i, 0)),
                   pl.BlockSpec((NB * B, C), lambda i: (i, 0))],
        out_shape=[jax.ShapeDtypeStruct((M * B, C), jnp.float32),
                   jax.ShapeDtypeStruct((M * B, C), jnp.int32)],
    )(x)


def _unpool4(y, idx, M, C):
    NB = 128
    while M % NB:
        NB //= 2

    def body(y_ref, i_ref, o_ref):
        yr = y_ref[...].reshape(NB, 1, B, C)
        ir = i_ref[...].reshape(NB, 1, B, C)
        j = lax.broadcasted_iota(jnp.int32, (NB, 4, B, C), 1)
        out = jnp.where(ir == j, yr, 0.0)
        o_ref[...] = out.reshape(NB * 4 * B, C)

    return pl.pallas_call(
        body,
        grid=(M // NB,),
        in_specs=[pl.BlockSpec((NB * B, C), lambda i: (i, 0)),
                  pl.BlockSpec((NB * B, C), lambda i: (i, 0))],
        out_specs=pl.BlockSpec((NB * 4 * B, C), lambda i: (i, 0)),
        out_shape=jax.ShapeDtypeStruct((M * 4 * B, C), jnp.float32),
    )(y, idx)


def _split3(W):
    return jnp.split(W, 3, axis=0)


def _cheb(xs, src, w, Ws, N, stats=True, bias8=None, extra=()):
    terms = []
    for x, Wi in zip(xs, Ws):
        W0, W1, W2 = _split3(Wi)
        cin = x.shape[1]
        xv = x.reshape(N, B * cin)
        x1 = _lap(xv, src, w)
        x2 = _lap_sub(x1, src, w, xv).reshape(N * B, cin)
        x1 = x1.reshape(N * B, cin)
        terms += [(x, W0), (x1, W1), (x2, W2)]
    return _mm(terms, list(extra), bias8=bias8, stats=stats)


def kernel(x, params, lap_w, edges):
    p = params
    s0, w0 = edges["src0"], lap_w["w0"]
    s1, w1 = edges["src1"], lap_w["w1"]
    s2, w2 = edges["src2"], lap_w["w2"]

    xa = jnp.transpose(x, (2, 0, 1, 3)).reshape(N0, B, 14)
    xa = jnp.pad(xa, ((0, 0), (0, 0), (0, 18))).reshape(N0 * B, 32)

    padW = lambda W: jnp.pad(W, ((0, 18), (0, 0)))

    def splitW(W, cs):
        W0, W1, W2 = _split3(W)
        out, o = [], 0
        for c in cs:
            out.append(jnp.concatenate(
                [W0[o:o + c], W1[o:o + c], W2[o:o + c]], axis=0))
            o += c
        return out

    W0, W1, W2 = _split3(p["c11"]["W"])
    Wc11 = jnp.concatenate([padW(W0), padW(W1), padW(W2)], axis=0)
    y, st = _cheb([xa], s0, w0, [Wc11], N0)
    e11 = _mm([], [("bnrelu", y,
                    _affine8(st, p["c11"]["g"], p["c11"]["be"], N0 * B))])

    y, st = _cheb([e11], s0, w0, [p["c13"]["W"]], N0)
    aff = _affine8(st, p["c13"]["g"], p["c13"]["be"], N0 * B)
    e1 = _mm([(xa, padW(p["r1"]["W"]))], [("bnrelu", y, aff)],
             bias8=_bias8(p["r1"]["b"]))

    e2i, idx1 = _pool4(e1, N0, 128)

    y, st = _cheb([e2i], s1, w1, [p["c21"]["W"]], N1)
    e2a = _mm([], [("bnrelu", y,
                    _affine8(st, p["c21"]["g"], p["c21"]["be"], N1 * B))])
    y, st = _cheb([e2a], s1, w1, [p["c23"]["W"]], N1)
    aff = _affine8(st, p["c23"]["g"], p["c23"]["be"], N1 * B)
    e2 = _mm([(e2i, p["r2"]["W"])], [("bnrelu", y, aff)],
             bias8=_bias8(p["r2"]["b"]))

    e3i, idx2 = _pool4(e2, N1, 256)

    y, st = _cheb([e3i], s2, w2, [p["c31"]["W"]], N2)
    e3a = _mm([], [("bnrelu", y,
                    _affine8(st, p["c31"]["g"], p["c31"]["be"], N2 * B))])
    y, st = _cheb([e3a[:, :256], e3a[:, 256:]], s2, w2,
                  splitW(p["c33"]["W"], [256, 256]), N2)
    aff = _affine8(st, p["c33"]["g"], p["c33"]["be"], N2 * B)
    e3 = _mm([(e3i, p["r3"]["W"])], [("bnrelu", y, aff)],
             bias8=_bias8(p["r3"]["b"]))

    u2 = _unpool4(e3, idx2, N2, 256)
    y, st = _cheb([u2, e2], s1, w1, splitW(p["u21"]["W"], [256, 256]), N1)
    y21 = _mm([], [("bnrelu", y,
                    _affine8(st, p["u21"]["g"], p["u21"]["be"], N1 * B))])
    y, st = _cheb([y21], s1, w1, [p["u22"]["W"]], N1)
    y22 = _mm([], [("bnrelu", y,
                    _affine8(st, p["u22"]["g"], p["u22"]["be"], N1 * B))])

    u1 = _unpool4(y22, idx1, N1, 128)
    y, st = _cheb([u1, e1], s0, w0, splitW(p["u11"]["W"], [128, 128]), N0)
    y11 = _mm([], [("bnrelu", y,
                    _affine8(st, p["u11"]["g"], p["u11"]["be"], N0 * B))])
    y, st = _cheb([y11], s0, w0, [p["u12"]["W"]], N0)
    y12 = _mm([], [("bnrelu", y,
                    _affine8(st, p["u12"]["g"], p["u12"]["be"], N0 * B))])

    xl = jnp.transpose(x[:, 1, :, 5:7], (1, 0, 2))
    xl4 = jnp.concatenate([xl, xl], axis=-1).reshape(N0 * B, 4)
    yf = _cheb([y12, e11], s0, w0, splitW(p["u13"]["W"], [64, 64]), N0,
               stats=False, bias8=_bias8(p["u13"]["b"]),
               extra=[("plain", xl4, 1.0)])

    out = yf.reshape(N0, B, 2, 2)
    return jnp.transpose(out, (1, 2, 0, 3))
